# Initial kernel scaffold; baseline (speedup 1.0000x reference)
import jax
import jax.numpy as jnp
from jax.experimental import pallas as pl


def kernel(feat, edge_index, W_src, b_src, W_dst, b_dst):
    raise NotImplementedError("write your pallas kernel here")


# speedup vs baseline: 3.9136x; 3.9136x over previous
import functools

import jax
import jax.numpy as jnp
from jax import lax
from jax.experimental import pallas as pl
from jax.experimental.pallas import tpu as pltpu
from jax.experimental.pallas import tpu_sc as plsc

N_USERS = 5000
N_ITEMS = 5000
E = 320000
D = 128

NP = 5120
NC, NS, L = 2, 16, 16
NW = NC * NS
EP = 327680
EROWS = EP // D
E_FULL_ROWS = E // D


_BS = 512


def _scores_body(a_ref, b_ref, o_ref):
    o_ref[...] = lax.dot_general(
        a_ref[...], b_ref[...], (((1,), (1,)), ((), ())),
        preferred_element_type=jnp.float32,
        precision=lax.Precision.HIGHEST,
    ) * jnp.float32(1.0 / jnp.sqrt(jnp.float32(D)))


def _scores(a, b):
    grid = (NP // _BS, NP // _BS)
    return pl.pallas_call(
        _scores_body,
        grid=grid,
        in_specs=[
            pl.BlockSpec((_BS, D), lambda i, j: (i, 0)),
            pl.BlockSpec((_BS, D), lambda i, j: (j, 0)),
        ],
        out_specs=pl.BlockSpec((_BS, _BS), lambda i, j: (i, j)),
        out_shape=jax.ShapeDtypeStruct((NP, NP), jnp.float32),
    )(a, b)


_FB = 5000


def _feat_body(x_ref, w_ref, b_ref, o_ref):
    y = lax.dot_general(
        x_ref[...], w_ref[0], (((1,), (1,)), ((), ())),
        preferred_element_type=jnp.float32,
        precision=lax.Precision.HIGHEST,
    ) + b_ref[0]
    o_ref[...] = jnp.maximum(y, 0.0)


def _features(feat, w_stack, b_stack):
    n_half_blocks = N_USERS // _FB
    grid = ((N_USERS + N_ITEMS) // _FB,)
    return pl.pallas_call(
        _feat_body,
        grid=grid,
        in_specs=[
            pl.BlockSpec((_FB, D), lambda i: (i, 0)),
            pl.BlockSpec((1, D, D), lambda i: (i // n_half_blocks, 0, 0)),
            pl.BlockSpec((1, 1, D), lambda i: (i // n_half_blocks, 0, 0)),
        ],
        out_specs=pl.BlockSpec((_FB, D), lambda i: (i, 0)),
        out_shape=jax.ShapeDtypeStruct((N_USERS + N_ITEMS, D), jnp.float32),
    )(feat, w_stack, b_stack)


def _softmax_body(a_ref, o_ref):
    a = a_ref[...]
    row = lax.broadcasted_iota(jnp.int32, a.shape, 0)
    a = jnp.where(row < E_FULL_ROWS, a, jnp.float32(-1e30))
    m = jnp.max(a)
    e = jnp.exp(a - m)
    o_ref[...] = e * (1.0 / jnp.sum(e))


def _softmax(alpha2d):
    return pl.pallas_call(
        _softmax_body,
        out_shape=jax.ShapeDtypeStruct((EROWS, D), jnp.float32),
    )(alpha2d)


_ACH = 128
_ANCH = EP // NW // _ACH


@functools.lru_cache(maxsize=None)
def _make_alpha_kernel():
    mesh = plsc.VectorSubcoreMesh(core_axis_name="c", subcore_axis_name="s",
                                  num_cores=NC, num_subcores=NS)
    return pl.kernel(
        _alpha_body,
        out_type=jax.ShapeDtypeStruct((EROWS, D), jnp.float32),
        mesh=mesh,
        scratch_types=[
            pltpu.VMEM((_ANCH, _ACH), jnp.int32),
            pltpu.VMEM((_ANCH, _ACH), jnp.int32),
            pltpu.VMEM((_ANCH, _ACH), jnp.int32),
            pltpu.VMEM((_ANCH, _ACH), jnp.float32),
            pltpu.SemaphoreType.DMA,
        ],
    )


def _alpha_body(s_hbm, u_hbm, i_hbm, alpha_hbm, u_v, i_v, idx_v, a_v, sem):
    wid = lax.axis_index("s") * NC + lax.axis_index("c")
    row0 = wid * _ANCH
    pltpu.sync_copy(u_hbm.at[pl.ds(row0, _ANCH)], u_v)
    pltpu.sync_copy(i_hbm.at[pl.ds(row0, _ANCH)], i_v)

    def flat_row(j, carry):
        for t in range(_ACH // L):
            sl = (j, pl.ds(t * L, L))
            idx_v[sl] = u_v[sl] * NP + i_v[sl]
        return carry

    lax.fori_loop(0, _ANCH, flat_row, 0)

    def gather_row(j, carry):
        pltpu.async_copy(s_hbm.at[idx_v.at[j]], a_v.at[j], sem).wait()
        return carry

    lax.fori_loop(0, _ANCH, gather_row, 0)
    pltpu.sync_copy(a_v, alpha_hbm.at[pl.ds(row0, _ANCH)])


_MCH = 128
_MNCH = EP // NS // _MCH
_AROWS = NP // NS


@functools.lru_cache(maxsize=None)
def _make_msg_kernel():
    mesh = plsc.VectorSubcoreMesh(core_axis_name="c", subcore_axis_name="s",
                                  num_cores=NC, num_subcores=NS)
    return pl.kernel(
        _msg_body,
        out_type=jax.ShapeDtypeStruct((2, NP, D), jnp.float32),
        mesh=mesh,
        scratch_types=[
            pltpu.VMEM((_MNCH, _MCH), jnp.int32),
            pltpu.VMEM((_MNCH, _MCH), jnp.int32),
            pltpu.VMEM((_MNCH, _MCH), jnp.float32),
            pltpu.VMEM((_MCH, D), jnp.float32),
            pltpu.VMEM_SHARED((NP, D), jnp.float32),
            pltpu.SemaphoreType.DMA,
        ],
    )


def _msg_body(g_hbm, gidx_hbm, sidx_hbm, w_hbm, zeros_hbm, out_hbm,
              gi_v, si_v, w_v, rows_v, acc_sh, sem):
    c = lax.axis_index("c")
    s = lax.axis_index("s")
    pltpu.sync_copy(zeros_hbm.at[pl.ds(s * _AROWS, _AROWS)],
                    acc_sh.at[pl.ds(s * _AROWS, _AROWS)])
    plsc.subcore_barrier()

    base = s * _MNCH
    pltpu.sync_copy(gidx_hbm.at[c, pl.ds(base, _MNCH)], gi_v)
    pltpu.sync_copy(sidx_hbm.at[c, pl.ds(base, _MNCH)], si_v)
    pltpu.sync_copy(w_hbm.at[pl.ds(base, _MNCH)], w_v)

    def chunk(j, carry):
        pltpu.async_copy(g_hbm.at[gi_v.at[j]], rows_v, sem).wait()

        def group(g, inner):
            wv = w_v[j, pl.ds(g * L, L)]
            for r16 in range(L):
                ws = wv[r16]
                e = g * L + r16
                for t in range(D // L):
                    sl = (e, pl.ds(t * L, L))
                    rows_v[sl] = rows_v[sl] * ws
            return inner

        lax.fori_loop(0, _MCH // L, group, 0)
        pltpu.sync_copy(rows_v, acc_sh.at[si_v.at[j]], add=True)
        return carry

    lax.fori_loop(0, _MNCH, chunk, 0)
    plsc.subcore_barrier()
    pltpu.sync_copy(acc_sh.at[pl.ds(s * _AROWS, _AROWS)],
                    out_hbm.at[c, pl.ds(s * _AROWS, _AROWS)])


def kernel(feat, edge_index, W_src, b_src, W_dst, b_dst):
    h_src = feat[:N_USERS]
    h_dst = feat[N_USERS:]
    u = edge_index[0]
    i = edge_index[1]

    h_src_p = jnp.pad(h_src, ((0, NP - N_USERS), (0, 0)))
    h_dst_p = jnp.pad(h_dst, ((0, NP - N_ITEMS), (0, 0)))
    u_p = jnp.pad(u, (0, EP - E)).reshape(EROWS, D)
    i_p = jnp.pad(i, (0, EP - E)).reshape(EROWS, D)

    s_mat = _scores(h_src_p, h_dst_p)
    w_stack = jnp.stack([W_src, W_dst])
    b_stack = jnp.stack([b_src, b_dst]).reshape(2, 1, D)
    feats = _features(feat, w_stack, b_stack)

    alpha2d = _make_alpha_kernel()(s_mat.reshape(NP * NP), u_p, i_p)

    w2d = _softmax(alpha2d)

    gidx = jnp.stack([u_p, i_p + N_USERS])
    sidx = jnp.stack([i_p, u_p])
    zeros_acc = jnp.zeros((NP, D), jnp.float32)
    out = _make_msg_kernel()(feats, gidx, sidx, w2d, zeros_acc)

    item_new = out[0, :N_ITEMS]
    user_new = out[1, :N_USERS]
    return jnp.concatenate([user_new, item_new], axis=0)


# speedup vs baseline: 4.8501x; 1.2393x over previous
import functools

import jax
import jax.numpy as jnp
from jax import lax
from jax.experimental import pallas as pl
from jax.experimental.pallas import tpu as pltpu
from jax.experimental.pallas import tpu_sc as plsc

N_USERS = 5000
N_ITEMS = 5000
E = 320000
D = 128

NP = 5120
NC, NS, L = 2, 16, 16
NW = NC * NS
EP = 327680
EROWS = EP // D
E_FULL_ROWS = E // D


_BS = 512


def _scores_body(a_ref, b_ref, o_ref):
    o_ref[...] = lax.dot_general(
        a_ref[...], b_ref[...], (((1,), (1,)), ((), ())),
        preferred_element_type=jnp.float32,
        precision=lax.Precision.HIGHEST,
    ) * jnp.float32(1.0 / jnp.sqrt(jnp.float32(D)))


def _scores(a, b):
    grid = (NP // _BS, NP // _BS)
    return pl.pallas_call(
        _scores_body,
        grid=grid,
        in_specs=[
            pl.BlockSpec((_BS, D), lambda i, j: (i, 0)),
            pl.BlockSpec((_BS, D), lambda i, j: (j, 0)),
        ],
        out_specs=pl.BlockSpec((_BS, _BS), lambda i, j: (i, j)),
        out_shape=jax.ShapeDtypeStruct((NP, NP), jnp.float32),
    )(a, b)


_FB = 5000


def _feat_body(x_ref, w_ref, b_ref, o_ref):
    y = lax.dot_general(
        x_ref[...], w_ref[0], (((1,), (1,)), ((), ())),
        preferred_element_type=jnp.float32,
        precision=lax.Precision.HIGHEST,
    ) + b_ref[0]
    o_ref[...] = jnp.maximum(y, 0.0)


def _features(feat, w_stack, b_stack):
    n_half_blocks = N_USERS // _FB
    grid = ((N_USERS + N_ITEMS) // _FB,)
    return pl.pallas_call(
        _feat_body,
        grid=grid,
        in_specs=[
            pl.BlockSpec((_FB, D), lambda i: (i, 0)),
            pl.BlockSpec((1, D, D), lambda i: (i // n_half_blocks, 0, 0)),
            pl.BlockSpec((1, 1, D), lambda i: (i // n_half_blocks, 0, 0)),
        ],
        out_specs=pl.BlockSpec((_FB, D), lambda i: (i, 0)),
        out_shape=jax.ShapeDtypeStruct((N_USERS + N_ITEMS, D), jnp.float32),
    )(feat, w_stack, b_stack)


def _softmax_body(a_ref, o_ref):
    a = a_ref[...]
    row = lax.broadcasted_iota(jnp.int32, a.shape, 0)
    a = jnp.where(row < E_FULL_ROWS, a, jnp.float32(-1e30))
    m = jnp.max(a)
    e = jnp.exp(a - m)
    o_ref[...] = e * (1.0 / jnp.sum(e))


def _softmax(alpha2d):
    return pl.pallas_call(
        _softmax_body,
        out_shape=jax.ShapeDtypeStruct((EROWS, D), jnp.float32),
    )(alpha2d)


_ACH = 128
_ANCH = EP // NW // _ACH


@functools.lru_cache(maxsize=None)
def _make_alpha_kernel():
    mesh = plsc.VectorSubcoreMesh(core_axis_name="c", subcore_axis_name="s",
                                  num_cores=NC, num_subcores=NS)
    return pl.kernel(
        _alpha_body,
        out_type=jax.ShapeDtypeStruct((EROWS, D), jnp.float32),
        mesh=mesh,
        scratch_types=[
            pltpu.VMEM((_ANCH, _ACH), jnp.int32),
            pltpu.VMEM((_ANCH, _ACH), jnp.int32),
            pltpu.VMEM((_ANCH, _ACH), jnp.int32),
            pltpu.VMEM((_ANCH, _ACH), jnp.float32),
            pltpu.SemaphoreType.DMA,
        ],
    )


def _alpha_body(s_hbm, u_hbm, i_hbm, alpha_hbm, u_v, i_v, idx_v, a_v, sem):
    wid = lax.axis_index("s") * NC + lax.axis_index("c")
    row0 = wid * _ANCH
    pltpu.sync_copy(u_hbm.at[pl.ds(row0, _ANCH)], u_v)
    pltpu.sync_copy(i_hbm.at[pl.ds(row0, _ANCH)], i_v)

    def flat_row(j, carry):
        for t in range(_ACH // L):
            sl = (j, pl.ds(t * L, L))
            idx_v[sl] = u_v[sl] * NP + i_v[sl]
        return carry

    lax.fori_loop(0, _ANCH, flat_row, 0)

    def gather_start(j, carry):
        pltpu.async_copy(s_hbm.at[idx_v.at[j]], a_v.at[j], sem)
        return carry

    lax.fori_loop(0, _ANCH, gather_start, 0)

    def gather_drain(j, carry):
        pltpu.make_async_copy(s_hbm.at[idx_v.at[j]], a_v.at[j], sem).wait()
        return carry

    lax.fori_loop(0, _ANCH, gather_drain, 0)
    pltpu.sync_copy(a_v, alpha_hbm.at[pl.ds(row0, _ANCH)])


_MCH = 128
_MNCH = EP // NS // _MCH
_AROWS = NP // NS
_NBUF = 4
_MPH = 4
_PCH = _MNCH // _MPH


@functools.lru_cache(maxsize=None)
def _make_msg_kernel():
    mesh = plsc.VectorSubcoreMesh(core_axis_name="c", subcore_axis_name="s",
                                  num_cores=NC, num_subcores=NS)
    return pl.kernel(
        _msg_body,
        out_type=jax.ShapeDtypeStruct((2, NP, D), jnp.float32),
        mesh=mesh,
        scratch_types=[
            pltpu.VMEM((_PCH, _MCH), jnp.int32),
            pltpu.VMEM((_PCH, _MCH), jnp.int32),
            pltpu.VMEM((_PCH, _MCH), jnp.float32),
            pltpu.VMEM((_NBUF, _MCH, D), jnp.float32),
            pltpu.VMEM_SHARED((NP, D), jnp.float32),
            pltpu.SemaphoreType.DMA,
            pltpu.SemaphoreType.DMA,
            pltpu.SemaphoreType.DMA,
            pltpu.SemaphoreType.DMA,
            pltpu.SemaphoreType.DMA,
            pltpu.SemaphoreType.DMA,
            pltpu.SemaphoreType.DMA,
            pltpu.SemaphoreType.DMA,
        ],
    )


def _msg_body(g_hbm, gidx_hbm, sidx_hbm, w_hbm, zeros_hbm, out_hbm,
              gi_v, si_v, w_v, rows_v, acc_sh,
              sg0, sg1, sg2, sg3, ss0, ss1, ss2, ss3):
    semg = (sg0, sg1, sg2, sg3)
    sems = (ss0, ss1, ss2, ss3)
    c = lax.axis_index("c")
    s = lax.axis_index("s")
    pltpu.sync_copy(zeros_hbm.at[pl.ds(s * _AROWS, _AROWS)],
                    acc_sh.at[pl.ds(s * _AROWS, _AROWS)])
    plsc.subcore_barrier()

    def g_start(j, b):
        pltpu.async_copy(g_hbm.at[gi_v.at[j]], rows_v.at[b], semg[b])

    def g_wait(j, b):
        pltpu.make_async_copy(g_hbm.at[gi_v.at[j]], rows_v.at[b], semg[b]).wait()

    def s_start(j, b):
        pltpu.async_copy(rows_v.at[b], acc_sh.at[si_v.at[j]], sems[b], add=True)

    def s_wait(j, b):
        pltpu.make_async_copy(rows_v.at[b], acc_sh.at[si_v.at[j]],
                              sems[b]).wait()

    def scale(j, b):
        def group(g, inner):
            wv = w_v[j, pl.ds(g * L, L)]
            for r16 in range(L):
                ws = wv[r16]
                e = g * L + r16
                for t in range(D // L):
                    sl = (b, e, pl.ds(t * L, L))
                    rows_v[sl] = rows_v[sl] * ws
            return inner

        lax.fori_loop(0, _MCH // L, group, 0)

    def phase(p, carry):
        pbase = s * _MNCH + p * _PCH
        pltpu.sync_copy(gidx_hbm.at[c, pl.ds(pbase, _PCH)], gi_v)
        pltpu.sync_copy(sidx_hbm.at[c, pl.ds(pbase, _PCH)], si_v)
        pltpu.sync_copy(w_hbm.at[pl.ds(pbase, _PCH)], w_v)
        g_start(0, 0)
        g_start(1, 1)

        def outer(m, carry2):
            for k in range(_NBUF):
                j = m * _NBUF + k
                g_wait(j, k)
                scale(j, k)
                s_start(j, k)
                kn = (k + 2) % _NBUF
                if k >= 2:
                    s_wait(j - 2, kn)
                else:
                    @pl.when(m >= 1)
                    def _():
                        s_wait(j - 2, kn)

                g_start(lax.rem(j + 2, _PCH), kn)
            return carry2

        lax.fori_loop(0, _PCH // _NBUF, outer, 0)
        g_wait(0, 0)
        g_wait(1, 1)
        s_wait(_PCH - 2, 2)
        s_wait(_PCH - 1, 3)
        return carry

    lax.fori_loop(0, _MPH, phase, 0)
    plsc.subcore_barrier()
    pltpu.sync_copy(acc_sh.at[pl.ds(s * _AROWS, _AROWS)],
                    out_hbm.at[c, pl.ds(s * _AROWS, _AROWS)])


def kernel(feat, edge_index, W_src, b_src, W_dst, b_dst):
    h_src = feat[:N_USERS]
    h_dst = feat[N_USERS:]
    u = edge_index[0]
    i = edge_index[1]

    h_src_p = jnp.pad(h_src, ((0, NP - N_USERS), (0, 0)))
    h_dst_p = jnp.pad(h_dst, ((0, NP - N_ITEMS), (0, 0)))
    u_p = jnp.pad(u, (0, EP - E)).reshape(EROWS, D)
    i_p = jnp.pad(i, (0, EP - E)).reshape(EROWS, D)

    s_mat = _scores(h_src_p, h_dst_p)
    w_stack = jnp.stack([W_src, W_dst])
    b_stack = jnp.stack([b_src, b_dst]).reshape(2, 1, D)
    feats = _features(feat, w_stack, b_stack)

    alpha2d = _make_alpha_kernel()(s_mat.reshape(NP * NP), u_p, i_p)

    w2d = _softmax(alpha2d)

    gidx = jnp.stack([u_p, i_p + N_USERS])
    sidx = jnp.stack([i_p, u_p])
    zeros_acc = jnp.zeros((NP, D), jnp.float32)
    out = _make_msg_kernel()(feats, gidx, sidx, w2d, zeros_acc)

    item_new = out[0, :N_ITEMS]
    user_new = out[1, :N_USERS]
    return jnp.concatenate([user_new, item_new], axis=0)


# speedup vs baseline: 4.8541x; 1.0008x over previous
import functools

import jax
import jax.numpy as jnp
from jax import lax
from jax.experimental import pallas as pl
from jax.experimental.pallas import tpu as pltpu
from jax.experimental.pallas import tpu_sc as plsc

N_USERS = 5000
N_ITEMS = 5000
E = 320000
D = 128

NP = 5120
NC, NS, L = 2, 16, 16
NW = NC * NS
EP = 327680
EROWS = EP // D
E_FULL_ROWS = E // D


_BS = 512


def _scores_body(a_ref, b_ref, o_ref):
    o_ref[...] = lax.dot_general(
        a_ref[...], b_ref[...], (((1,), (1,)), ((), ())),
        preferred_element_type=jnp.float32,
        precision=lax.Precision.HIGHEST,
    ) * jnp.float32(1.0 / jnp.sqrt(jnp.float32(D)))


def _scores(a, b):
    grid = (NP // _BS, NP // _BS)
    return pl.pallas_call(
        _scores_body,
        grid=grid,
        in_specs=[
            pl.BlockSpec((_BS, D), lambda i, j: (i, 0)),
            pl.BlockSpec((_BS, D), lambda i, j: (j, 0)),
        ],
        out_specs=pl.BlockSpec((_BS, _BS), lambda i, j: (i, j)),
        out_shape=jax.ShapeDtypeStruct((NP, NP), jnp.float32),
    )(a, b)


_FB = 5000


def _feat_body(x_ref, w_ref, b_ref, o_ref):
    y = lax.dot_general(
        x_ref[...], w_ref[0], (((1,), (1,)), ((), ())),
        preferred_element_type=jnp.float32,
        precision=lax.Precision.HIGHEST,
    ) + b_ref[0]
    o_ref[...] = jnp.maximum(y, 0.0)


def _features(feat, w_stack, b_stack):
    n_half_blocks = N_USERS // _FB
    grid = ((N_USERS + N_ITEMS) // _FB,)
    return pl.pallas_call(
        _feat_body,
        grid=grid,
        in_specs=[
            pl.BlockSpec((_FB, D), lambda i: (i, 0)),
            pl.BlockSpec((1, D, D), lambda i: (i // n_half_blocks, 0, 0)),
            pl.BlockSpec((1, 1, D), lambda i: (i // n_half_blocks, 0, 0)),
        ],
        out_specs=pl.BlockSpec((_FB, D), lambda i: (i, 0)),
        out_shape=jax.ShapeDtypeStruct((N_USERS + N_ITEMS, D), jnp.float32),
    )(feat, w_stack, b_stack)


def _softmax_body(a_ref, o_ref):
    a = a_ref[...]
    row = lax.broadcasted_iota(jnp.int32, a.shape, 0)
    a = jnp.where(row < E_FULL_ROWS, a, jnp.float32(-1e30))
    m = jnp.max(a)
    e = jnp.exp(a - m)
    o_ref[...] = e * (1.0 / jnp.sum(e))


def _softmax(alpha2d):
    return pl.pallas_call(
        _softmax_body,
        out_shape=jax.ShapeDtypeStruct((EROWS, D), jnp.float32),
    )(alpha2d)


_ACH = 128
_ANCH = EP // NW // _ACH


@functools.lru_cache(maxsize=None)
def _make_alpha_kernel():
    mesh = plsc.VectorSubcoreMesh(core_axis_name="c", subcore_axis_name="s",
                                  num_cores=NC, num_subcores=NS)
    return pl.kernel(
        _alpha_body,
        out_type=jax.ShapeDtypeStruct((EROWS, D), jnp.float32),
        mesh=mesh,
        scratch_types=[
            pltpu.VMEM((_ANCH, _ACH), jnp.int32),
            pltpu.VMEM((_ANCH, _ACH), jnp.int32),
            pltpu.VMEM((_ANCH, _ACH), jnp.int32),
            pltpu.VMEM((_ANCH, _ACH), jnp.float32),
            pltpu.SemaphoreType.DMA,
        ],
    )


def _alpha_body(s_hbm, u_hbm, i_hbm, alpha_hbm, u_v, i_v, idx_v, a_v, sem):
    wid = lax.axis_index("s") * NC + lax.axis_index("c")
    row0 = wid * _ANCH
    pltpu.sync_copy(u_hbm.at[pl.ds(row0, _ANCH)], u_v)
    pltpu.sync_copy(i_hbm.at[pl.ds(row0, _ANCH)], i_v)

    def flat_row(j, carry):
        for t in range(_ACH // L):
            sl = (j, pl.ds(t * L, L))
            idx_v[sl] = u_v[sl] * NP + i_v[sl]
        return carry

    lax.fori_loop(0, _ANCH, flat_row, 0)

    def gather_start(j, carry):
        pltpu.async_copy(s_hbm.at[idx_v.at[j]], a_v.at[j], sem)
        return carry

    lax.fori_loop(0, _ANCH, gather_start, 0)

    def gather_drain(j, carry):
        pltpu.make_async_copy(s_hbm.at[idx_v.at[j]], a_v.at[j], sem).wait()
        return carry

    lax.fori_loop(0, _ANCH, gather_drain, 0)
    pltpu.sync_copy(a_v, alpha_hbm.at[pl.ds(row0, _ANCH)])


_MCH = 128
_MNCH = EP // NS // _MCH
_AROWS = NP // NS
_NBUF = 4
_MPH = 4
_PCH = _MNCH // _MPH


@functools.lru_cache(maxsize=None)
def _make_msg_kernel():
    mesh = plsc.VectorSubcoreMesh(core_axis_name="c", subcore_axis_name="s",
                                  num_cores=NC, num_subcores=NS)
    return pl.kernel(
        _msg_body,
        out_type=jax.ShapeDtypeStruct((2, NP, D), jnp.float32),
        mesh=mesh,
        scratch_types=[
            pltpu.VMEM((_PCH, _MCH), jnp.int32),
            pltpu.VMEM((_PCH, _MCH), jnp.int32),
            pltpu.VMEM((_PCH, _MCH), jnp.float32),
            pltpu.VMEM((_NBUF, _MCH, D), jnp.float32),
            pltpu.VMEM_SHARED((NP, D), jnp.float32),
            pltpu.SemaphoreType.DMA,
            pltpu.SemaphoreType.DMA,
            pltpu.SemaphoreType.DMA,
            pltpu.SemaphoreType.DMA,
            pltpu.SemaphoreType.DMA,
            pltpu.SemaphoreType.DMA,
            pltpu.SemaphoreType.DMA,
            pltpu.SemaphoreType.DMA,
        ],
    )


def _msg_body(g_hbm, gidx_hbm, sidx_hbm, w_hbm, zeros_hbm, out_hbm,
              gi_v, si_v, w_v, rows_v, acc_sh,
              sg0, sg1, sg2, sg3, ss0, ss1, ss2, ss3):
    semg = (sg0, sg1, sg2, sg3)
    sems = (ss0, ss1, ss2, ss3)
    c = lax.axis_index("c")
    s = lax.axis_index("s")
    pltpu.sync_copy(zeros_hbm.at[pl.ds(s * _AROWS, _AROWS)],
                    acc_sh.at[pl.ds(s * _AROWS, _AROWS)])
    plsc.subcore_barrier()

    def g_start(j, b):
        pltpu.async_copy(g_hbm.at[gi_v.at[j]], rows_v.at[b], semg[b])

    def g_wait(j, b):
        pltpu.make_async_copy(g_hbm.at[gi_v.at[j]], rows_v.at[b], semg[b]).wait()

    def s_start(j, b):
        pltpu.async_copy(rows_v.at[b], acc_sh.at[si_v.at[j]], sems[b], add=True)

    def s_wait(j, b):
        pltpu.make_async_copy(rows_v.at[b], acc_sh.at[si_v.at[j]],
                              sems[b]).wait()

    def scale(j, b):
        @plsc.parallel_loop(0, _MCH // L, unroll=2)
        def group(g):
            wv = w_v[j, pl.ds(g * L, L)]
            for r16 in range(L):
                ws = wv[r16]
                e = g * L + r16
                for t in range(D // L):
                    sl = (b, e, pl.ds(t * L, L))
                    rows_v[sl] = rows_v[sl] * ws

    def phase(p, carry):
        pbase = s * _MNCH + p * _PCH
        pltpu.sync_copy(gidx_hbm.at[c, pl.ds(pbase, _PCH)], gi_v)
        pltpu.sync_copy(sidx_hbm.at[c, pl.ds(pbase, _PCH)], si_v)
        pltpu.sync_copy(w_hbm.at[pl.ds(pbase, _PCH)], w_v)
        g_start(0, 0)
        g_start(1, 1)

        def outer(m, carry2):
            for k in range(_NBUF):
                j = m * _NBUF + k
                g_wait(j, k)
                scale(j, k)
                s_start(j, k)
                kn = (k + 2) % _NBUF
                if k >= 2:
                    s_wait(j - 2, kn)
                else:
                    @pl.when(m >= 1)
                    def _():
                        s_wait(j - 2, kn)

                g_start(lax.rem(j + 2, _PCH), kn)
            return carry2

        lax.fori_loop(0, _PCH // _NBUF, outer, 0)
        g_wait(0, 0)
        g_wait(1, 1)
        s_wait(_PCH - 2, 2)
        s_wait(_PCH - 1, 3)
        return carry

    lax.fori_loop(0, _MPH, phase, 0)
    plsc.subcore_barrier()
    pltpu.sync_copy(acc_sh.at[pl.ds(s * _AROWS, _AROWS)],
                    out_hbm.at[c, pl.ds(s * _AROWS, _AROWS)])


def kernel(feat, edge_index, W_src, b_src, W_dst, b_dst):
    h_src = feat[:N_USERS]
    h_dst = feat[N_USERS:]
    u = edge_index[0]
    i = edge_index[1]

    h_src_p = jnp.pad(h_src, ((0, NP - N_USERS), (0, 0)))
    h_dst_p = jnp.pad(h_dst, ((0, NP - N_ITEMS), (0, 0)))
    u_p = jnp.pad(u, (0, EP - E)).reshape(EROWS, D)
    i_p = jnp.pad(i, (0, EP - E)).reshape(EROWS, D)

    s_mat = _scores(h_src_p, h_dst_p)
    w_stack = jnp.stack([W_src, W_dst])
    b_stack = jnp.stack([b_src, b_dst]).reshape(2, 1, D)
    feats = _features(feat, w_stack, b_stack)

    alpha2d = _make_alpha_kernel()(s_mat.reshape(NP * NP), u_p, i_p)

    w2d = _softmax(alpha2d)

    gidx = jnp.stack([u_p, i_p + N_USERS])
    sidx = jnp.stack([i_p, u_p])
    zeros_acc = jnp.zeros((NP, D), jnp.float32)
    out = _make_msg_kernel()(feats, gidx, sidx, w2d, zeros_acc)

    item_new = out[0, :N_ITEMS]
    user_new = out[1, :N_USERS]
    return jnp.concatenate([user_new, item_new], axis=0)
